# baseline (speedup 1.0000x reference)
import functools

import jax
import jax.numpy as jnp
from jax import lax
from jax.experimental import pallas as pl
from jax.experimental.pallas import tpu as pltpu
from jax.experimental.pallas import tpu_sc as plsc

_NE = 1024
_D = 64
_BETA = 0.25
_N = 128 * 576

_TOK_BLOCK = 1024

_NSC = _N // 2
_NTC = _N - _NSC
_GSC = _NSC // _TOK_BLOCK
_GTC = _NTC // _TOK_BLOCK

_NC = 2
_NS = 16
_NW = _NC * _NS
_ROWS_PER_W = _NSC // _NW
_CHUNK = 128
_NCHUNK = _ROWS_PER_W // _CHUNK
_NBUF = 4


def _argmin_body(x_ref, e_ref, idx_ref, loss_ref):
    x = x_ref[...]
    e = e_ref[...]
    e2 = jnp.sum(e * e, axis=1, keepdims=True)
    x2 = jnp.sum(x * x, axis=1)
    es = e * (-2.0)
    prod = lax.dot_general(es, x, (((1,), (1,)), ((), ())),
                           preferred_element_type=jnp.float32)
    dist = prod + e2
    minval = jnp.min(dist, axis=0, keepdims=True)
    ids = lax.broadcasted_iota(jnp.int32, dist.shape, 0)
    idx = jnp.min(jnp.where(dist == minval, ids, _NE), axis=0)
    idx_ref[...] = idx.reshape(_TOK_BLOCK // 128, 128)

    @pl.when(pl.program_id(0) == 0)
    def _():
        loss_ref[...] = jnp.zeros((1, 1), jnp.float32)

    loss_ref[...] += (jnp.sum(minval) + jnp.sum(x2)).reshape(1, 1)


_argmin_call = pl.pallas_call(
    _argmin_body,
    grid=(_GSC,),
    in_specs=[
        pl.BlockSpec((_TOK_BLOCK, _D), lambda i: (i, 0)),
        pl.BlockSpec((_NE, _D), lambda i: (0, 0)),
    ],
    out_specs=[
        pl.BlockSpec((_TOK_BLOCK // 128, 128), lambda i: (i, 0)),
        pl.BlockSpec((1, 1), lambda i: (0, 0)),
    ],
    out_shape=[
        jax.ShapeDtypeStruct((_NSC // 128, 128), jnp.int32),
        jax.ShapeDtypeStruct((1, 1), jnp.float32),
    ],
)


def _argmin_zq_body(x_ref, e_ref, zq_ref, loss_ref):
    x = x_ref[...]
    e = e_ref[...]
    e2 = jnp.sum(e * e, axis=1, keepdims=True)
    x2 = jnp.sum(x * x, axis=1)
    es = e * (-2.0)
    prod = lax.dot_general(es, x, (((1,), (1,)), ((), ())),
                           preferred_element_type=jnp.float32)
    dist = prod + e2
    minval = jnp.min(dist, axis=0, keepdims=True)
    ids = lax.broadcasted_iota(jnp.int32, dist.shape, 0)
    idx = jnp.min(jnp.where(dist == minval, ids, _NE), axis=0)
    onehot = jnp.where(ids == idx[None, :], 1.0, 0.0)
    zq_ref[...] = lax.dot_general(onehot, e, (((0,), (0,)), ((), ())),
                                  preferred_element_type=jnp.float32)

    @pl.when(pl.program_id(0) == 0)
    def _():
        loss_ref[...] = jnp.zeros((1, 1), jnp.float32)

    loss_ref[...] += (jnp.sum(minval) + jnp.sum(x2)).reshape(1, 1)


_argmin_zq_call = pl.pallas_call(
    _argmin_zq_body,
    grid=(_GTC,),
    in_specs=[
        pl.BlockSpec((_TOK_BLOCK, _D), lambda i: (i, 0)),
        pl.BlockSpec((_NE, _D), lambda i: (0, 0)),
    ],
    out_specs=[
        pl.BlockSpec((_TOK_BLOCK, _D), lambda i: (i, 0)),
        pl.BlockSpec((1, 1), lambda i: (0, 0)),
    ],
    out_shape=[
        jax.ShapeDtypeStruct((_NTC, _D), jnp.float32),
        jax.ShapeDtypeStruct((1, 1), jnp.float32),
    ],
)


@functools.cache
def _make_gather_sc():
    def body(emb_hbm, idx3_hbm, out_hbm, idx_v, rows_v, gsem, ssem):
        wid = lax.axis_index("s") * _NC + lax.axis_index("c")
        base = wid * _ROWS_PER_W
        pltpu.sync_copy(idx3_hbm.at[pl.ds(wid * _NCHUNK, _NCHUNK)], idx_v)

        def gather(c):
            return pltpu.async_copy(
                emb_hbm.at[idx_v.at[c]], rows_v.at[c % _NBUF], gsem)

        def scatter(c):
            return pltpu.async_copy(
                rows_v.at[c % _NBUF],
                out_hbm.at[pl.ds(base + c * _CHUNK, _CHUNK)], ssem)

        gh = [None] * _NCHUNK
        sh = [None] * _NCHUNK
        for c in range(min(_NBUF - 1, _NCHUNK)):
            gh[c] = gather(c)
        for c in range(_NCHUNK):
            nxt = c + _NBUF - 1
            if nxt < _NCHUNK:
                if c >= 1:
                    sh[c - 1].wait()
                gh[nxt] = gather(nxt)
            gh[c].wait()
            sh[c] = scatter(c)
        for c in range(max(0, _NCHUNK - _NBUF + 1), _NCHUNK):
            sh[c].wait()

    return pl.kernel(
        body,
        out_type=jax.ShapeDtypeStruct((_NSC, _D), jnp.float32),
        mesh=plsc.VectorSubcoreMesh(core_axis_name="c", subcore_axis_name="s"),
        compiler_params=pltpu.CompilerParams(use_tc_tiling_on_sc=False),
        scratch_types=[
            pltpu.VMEM((_NCHUNK, _CHUNK), jnp.int32),
            pltpu.VMEM((_NBUF, _CHUNK, _D), jnp.float32),
            pltpu.SemaphoreType.DMA,
            pltpu.SemaphoreType.DMA,
        ],
    )


def kernel(z_e, embedding):
    flat = z_e.reshape(_N, _D)
    flat_sc = lax.slice(flat, (0, 0), (_NSC, _D))
    flat_tc = lax.slice(flat, (_NSC, 0), (_N, _D))
    idx2, loss_sc = _argmin_call(flat_sc, embedding)
    zq_sc = _make_gather_sc()(embedding, idx2)
    zq_tc, loss_tc = _argmin_zq_call(flat_tc, embedding)
    loss_acc = loss_sc + loss_tc
    z_q = jnp.concatenate([zq_sc, zq_tc], axis=0)
    loss = loss_acc[0, 0] * ((1.0 + _BETA) / (_N * _D))
    return z_q.reshape(z_e.shape), loss


# speedup vs baseline: 1.1834x; 1.1834x over previous
import functools

import jax
import jax.numpy as jnp
from jax import lax
from jax.experimental import pallas as pl
from jax.experimental.pallas import tpu as pltpu
from jax.experimental.pallas import tpu_sc as plsc

_NE = 1024
_D = 64
_BETA = 0.25
_N = 128 * 576

_TOK_BLOCK = 2048
_G = _N // _TOK_BLOCK

_NC = 2
_NS = 16
_NW = _NC * _NS
_ROWS_PER_W = _N // _NW
_CHUNK = 128
_NCHUNK = _ROWS_PER_W // _CHUNK
_NBUF = 4


def _argmin_body(x_ref, e_ref, idx_ref, loss_ref):
    x = x_ref[...]
    e = e_ref[...]
    e2 = jnp.sum(e * e, axis=1, keepdims=True)
    x2 = jnp.sum(x * x, axis=1)
    es = e * (-2.0)
    prod = lax.dot_general(es, x, (((1,), (1,)), ((), ())),
                           preferred_element_type=jnp.float32)
    dist = prod + e2
    minval = jnp.min(dist, axis=0, keepdims=True)
    ids = lax.broadcasted_iota(jnp.int32, dist.shape, 0)
    idx = jnp.min(jnp.where(dist == minval, ids, _NE), axis=0)
    idx_ref[...] = idx.reshape(_TOK_BLOCK // 128, 128)

    @pl.when(pl.program_id(0) == 0)
    def _():
        loss_ref[...] = jnp.zeros((1, 1), jnp.float32)

    loss_ref[...] += (jnp.sum(minval) + jnp.sum(x2)).reshape(1, 1)


_argmin_call = pl.pallas_call(
    _argmin_body,
    grid=(_G,),
    in_specs=[
        pl.BlockSpec((_TOK_BLOCK, _D), lambda i: (i, 0)),
        pl.BlockSpec((_NE, _D), lambda i: (0, 0)),
    ],
    out_specs=[
        pl.BlockSpec((_TOK_BLOCK // 128, 128), lambda i: (i, 0)),
        pl.BlockSpec((1, 1), lambda i: (0, 0)),
    ],
    out_shape=[
        jax.ShapeDtypeStruct((_N // 128, 128), jnp.int32),
        jax.ShapeDtypeStruct((1, 1), jnp.float32),
    ],
)


@functools.cache
def _make_gather_sc():
    def body(emb_hbm, idx3_hbm, out_hbm, idx_v, rows_v, gsem, ssem):
        wid = lax.axis_index("s") * _NC + lax.axis_index("c")
        base = wid * _ROWS_PER_W
        pltpu.sync_copy(idx3_hbm.at[pl.ds(wid * _NCHUNK, _NCHUNK)], idx_v)

        def gather(c):
            return pltpu.async_copy(
                emb_hbm.at[idx_v.at[c]], rows_v.at[c % _NBUF], gsem)

        def scatter(c):
            return pltpu.async_copy(
                rows_v.at[c % _NBUF],
                out_hbm.at[pl.ds(base + c * _CHUNK, _CHUNK)], ssem)

        gh = [None] * _NCHUNK
        sh = [None] * _NCHUNK
        for c in range(min(_NBUF - 1, _NCHUNK)):
            gh[c] = gather(c)
        for c in range(_NCHUNK):
            nxt = c + _NBUF - 1
            if nxt < _NCHUNK:
                if c >= 1:
                    sh[c - 1].wait()
                gh[nxt] = gather(nxt)
            gh[c].wait()
            sh[c] = scatter(c)
        for c in range(max(0, _NCHUNK - _NBUF + 1), _NCHUNK):
            sh[c].wait()

    return pl.kernel(
        body,
        out_type=jax.ShapeDtypeStruct((_N, _D), jnp.float32),
        mesh=plsc.VectorSubcoreMesh(core_axis_name="c", subcore_axis_name="s"),
        compiler_params=pltpu.CompilerParams(use_tc_tiling_on_sc=False),
        scratch_types=[
            pltpu.VMEM((_NCHUNK, _CHUNK), jnp.int32),
            pltpu.VMEM((_NBUF, _CHUNK, _D), jnp.float32),
            pltpu.SemaphoreType.DMA,
            pltpu.SemaphoreType.DMA,
        ],
    )


def kernel(z_e, embedding):
    flat = z_e.reshape(_N, _D)
    idx2, loss_acc = _argmin_call(flat, embedding)
    z_q = _make_gather_sc()(embedding, idx2)
    loss = loss_acc[0, 0] * ((1.0 + _BETA) / (_N * _D))
    return z_q.reshape(z_e.shape), loss


# speedup vs baseline: 1.2153x; 1.0269x over previous
import functools

import jax
import jax.numpy as jnp
from jax import lax
from jax.experimental import pallas as pl
from jax.experimental.pallas import tpu as pltpu
from jax.experimental.pallas import tpu_sc as plsc

_NE = 1024
_D = 64
_BETA = 0.25
_N = 128 * 576

_TOK_BLOCK = 4096
_G = _N // _TOK_BLOCK

_NC = 2
_NS = 16
_NW = _NC * _NS
_ROWS_PER_W = _N // _NW
_CHUNK = 128
_NCHUNK = _ROWS_PER_W // _CHUNK
_NBUF = 4


def _argmin_body(x_ref, e_ref, idx_ref, loss_ref):
    x = x_ref[...]
    e = e_ref[...]
    e2 = jnp.sum(e * e, axis=1, keepdims=True)
    x2 = jnp.sum(x * x, axis=1)
    es = e * (-2.0)
    prod = lax.dot_general(es, x, (((1,), (1,)), ((), ())),
                           preferred_element_type=jnp.float32)
    dist = prod + e2
    minval = jnp.min(dist, axis=0, keepdims=True)
    ids = lax.broadcasted_iota(jnp.int32, dist.shape, 0)
    idx = jnp.min(jnp.where(dist == minval, ids, _NE), axis=0)
    idx_ref[...] = idx.reshape(_TOK_BLOCK // 128, 128)

    @pl.when(pl.program_id(0) == 0)
    def _():
        loss_ref[...] = jnp.zeros((1, 1), jnp.float32)

    loss_ref[...] += (jnp.sum(minval) + jnp.sum(x2)).reshape(1, 1)


_argmin_call = pl.pallas_call(
    _argmin_body,
    grid=(_G,),
    in_specs=[
        pl.BlockSpec((_TOK_BLOCK, _D), lambda i: (i, 0)),
        pl.BlockSpec((_NE, _D), lambda i: (0, 0)),
    ],
    out_specs=[
        pl.BlockSpec((_TOK_BLOCK // 128, 128), lambda i: (i, 0)),
        pl.BlockSpec((1, 1), lambda i: (0, 0)),
    ],
    out_shape=[
        jax.ShapeDtypeStruct((_N // 128, 128), jnp.int32),
        jax.ShapeDtypeStruct((1, 1), jnp.float32),
    ],
)


@functools.cache
def _make_gather_sc():
    def body(emb_hbm, idx3_hbm, out_hbm, idx_v, rows_v, gsem, ssem):
        wid = lax.axis_index("s") * _NC + lax.axis_index("c")
        base = wid * _ROWS_PER_W
        pltpu.sync_copy(idx3_hbm.at[pl.ds(wid * _NCHUNK, _NCHUNK)], idx_v)

        def gather(c):
            return pltpu.async_copy(
                emb_hbm.at[idx_v.at[c]], rows_v.at[c % _NBUF], gsem)

        def scatter(c):
            return pltpu.async_copy(
                rows_v.at[c % _NBUF],
                out_hbm.at[pl.ds(base + c * _CHUNK, _CHUNK)], ssem)

        gh = [None] * _NCHUNK
        sh = [None] * _NCHUNK
        for c in range(min(_NBUF - 1, _NCHUNK)):
            gh[c] = gather(c)
        for c in range(_NCHUNK):
            nxt = c + _NBUF - 1
            if nxt < _NCHUNK:
                if c >= 1:
                    sh[c - 1].wait()
                gh[nxt] = gather(nxt)
            gh[c].wait()
            sh[c] = scatter(c)
        for c in range(max(0, _NCHUNK - _NBUF + 1), _NCHUNK):
            sh[c].wait()

    return pl.kernel(
        body,
        out_type=jax.ShapeDtypeStruct((_N, _D), jnp.float32),
        mesh=plsc.VectorSubcoreMesh(core_axis_name="c", subcore_axis_name="s"),
        compiler_params=pltpu.CompilerParams(use_tc_tiling_on_sc=False),
        scratch_types=[
            pltpu.VMEM((_NCHUNK, _CHUNK), jnp.int32),
            pltpu.VMEM((_NBUF, _CHUNK, _D), jnp.float32),
            pltpu.SemaphoreType.DMA,
            pltpu.SemaphoreType.DMA,
        ],
    )


def kernel(z_e, embedding):
    flat = z_e.reshape(_N, _D)
    idx2, loss_acc = _argmin_call(flat, embedding)
    z_q = _make_gather_sc()(embedding, idx2)
    loss = loss_acc[0, 0] * ((1.0 + _BETA) / (_N * _D))
    return z_q.reshape(z_e.shape), loss


# speedup vs baseline: 1.2260x; 1.0088x over previous
import functools

import jax
import jax.numpy as jnp
from jax import lax
from jax.experimental import pallas as pl
from jax.experimental.pallas import tpu as pltpu
from jax.experimental.pallas import tpu_sc as plsc

_NE = 1024
_D = 64
_BETA = 0.25
_N = 128 * 576

_TOK_BLOCK = 8192
_G = _N // _TOK_BLOCK

_NC = 2
_NS = 16
_NW = _NC * _NS
_ROWS_PER_W = _N // _NW
_CHUNK = 128
_NCHUNK = _ROWS_PER_W // _CHUNK
_NBUF = 4


def _argmin_body(x_ref, e_ref, idx_ref, loss_ref):
    x = x_ref[...]
    e = e_ref[...]
    e2 = jnp.sum(e * e, axis=1, keepdims=True)
    x2 = jnp.sum(x * x, axis=1)
    es = e * (-2.0)
    prod = lax.dot_general(es, x, (((1,), (1,)), ((), ())),
                           preferred_element_type=jnp.float32)
    dist = prod + e2
    minval = jnp.min(dist, axis=0, keepdims=True)
    ids = lax.broadcasted_iota(jnp.int32, dist.shape, 0)
    idx = jnp.min(jnp.where(dist == minval, ids, _NE), axis=0)
    idx_ref[...] = idx.reshape(_TOK_BLOCK // 128, 128)

    @pl.when(pl.program_id(0) == 0)
    def _():
        loss_ref[...] = jnp.zeros((1, 1), jnp.float32)

    loss_ref[...] += (jnp.sum(minval) + jnp.sum(x2)).reshape(1, 1)


_argmin_call = pl.pallas_call(
    _argmin_body,
    grid=(_G,),
    in_specs=[
        pl.BlockSpec((_TOK_BLOCK, _D), lambda i: (i, 0)),
        pl.BlockSpec((_NE, _D), lambda i: (0, 0)),
    ],
    out_specs=[
        pl.BlockSpec((_TOK_BLOCK // 128, 128), lambda i: (i, 0)),
        pl.BlockSpec((1, 1), lambda i: (0, 0)),
    ],
    out_shape=[
        jax.ShapeDtypeStruct((_N // 128, 128), jnp.int32),
        jax.ShapeDtypeStruct((1, 1), jnp.float32),
    ],
)


@functools.cache
def _make_gather_sc():
    def body(emb_hbm, idx3_hbm, out_hbm, idx_v, rows_v, gsem, ssem):
        wid = lax.axis_index("s") * _NC + lax.axis_index("c")
        base = wid * _ROWS_PER_W
        pltpu.sync_copy(idx3_hbm.at[pl.ds(wid * _NCHUNK, _NCHUNK)], idx_v)

        def gather(c):
            return pltpu.async_copy(
                emb_hbm.at[idx_v.at[c]], rows_v.at[c % _NBUF], gsem)

        def scatter(c):
            return pltpu.async_copy(
                rows_v.at[c % _NBUF],
                out_hbm.at[pl.ds(base + c * _CHUNK, _CHUNK)], ssem)

        gh = [None] * _NCHUNK
        sh = [None] * _NCHUNK
        for c in range(min(_NBUF - 1, _NCHUNK)):
            gh[c] = gather(c)
        for c in range(_NCHUNK):
            nxt = c + _NBUF - 1
            if nxt < _NCHUNK:
                if c >= 1:
                    sh[c - 1].wait()
                gh[nxt] = gather(nxt)
            gh[c].wait()
            sh[c] = scatter(c)
        for c in range(max(0, _NCHUNK - _NBUF + 1), _NCHUNK):
            sh[c].wait()

    return pl.kernel(
        body,
        out_type=jax.ShapeDtypeStruct((_N, _D), jnp.float32),
        mesh=plsc.VectorSubcoreMesh(core_axis_name="c", subcore_axis_name="s"),
        compiler_params=pltpu.CompilerParams(use_tc_tiling_on_sc=False),
        scratch_types=[
            pltpu.VMEM((_NCHUNK, _CHUNK), jnp.int32),
            pltpu.VMEM((_NBUF, _CHUNK, _D), jnp.float32),
            pltpu.SemaphoreType.DMA,
            pltpu.SemaphoreType.DMA,
        ],
    )


def kernel(z_e, embedding):
    flat = z_e.reshape(_N, _D)
    idx2, loss_acc = _argmin_call(flat, embedding)
    z_q = _make_gather_sc()(embedding, idx2)
    loss = loss_acc[0, 0] * ((1.0 + _BETA) / (_N * _D))
    return z_q.reshape(z_e.shape), loss
